# baseline (speedup 1.0000x reference)
import jax
import jax.numpy as jnp
from jax.experimental import pallas as pl
from jax.experimental.pallas import tpu as pltpu
from jax.experimental.pallas import tpu_sc as plsc

H = 768
F = 3072
E = 8
K = 2
T = 2048
A = T * K
BT = 256
NBLK = A // BT + E
NPAD = NBLK * BT
CH = 256
BTC = 512

def _vector_mesh():
    return plsc.VectorSubcoreMesh(core_axis_name="c", subcore_axis_name="s")


def _routing_body(x_ref, gw_ref, gb_ref, wgt_ref, pos_ref, blke_ref, blku_ref):
    x = x_ref[...]
    scores = (
        jnp.dot(x, gw_ref[...], preferred_element_type=jnp.float32)
        + gb_ref[...]
    )
    lane = jax.lax.broadcasted_iota(jnp.int32, (T, E), 1)
    idx1 = jnp.argmax(scores, axis=1)
    s1 = jnp.max(scores, axis=1)
    masked = jnp.where(lane == idx1[:, None], -jnp.inf, scores)
    idx2 = jnp.argmax(masked, axis=1)
    s2 = jnp.max(masked, axis=1)
    p1 = 1.0 / (1.0 + jnp.exp(s2 - s1))
    wgt_ref[...] = jnp.concatenate(
        [p1[:, None], (1.0 - p1)[:, None]], axis=1
    )

    oh1 = (lane == idx1[:, None]).astype(jnp.float32)
    oh2 = (lane == idx2[:, None]).astype(jnp.float32)
    sel = oh1 + oh2

    r = jax.lax.broadcasted_iota(jnp.int32, (CH, CH), 0)
    c = jax.lax.broadcasted_iota(jnp.int32, (CH, CH), 1)
    tri = (c < r).astype(jnp.float32)
    excl_chunks = []
    carry = jnp.zeros((1, E), jnp.float32)
    for ci in range(T // CH):
        blk = sel[ci * CH : (ci + 1) * CH, :]
        excl_chunks.append(
            jnp.dot(tri, blk, preferred_element_type=jnp.float32) + carry
        )
        carry = carry + jnp.sum(blk, axis=0, keepdims=True)
    excl = jnp.concatenate(excl_chunks, axis=0)
    counts = carry

    ci_ = counts.astype(jnp.int32)
    pc = ((ci_ + (BT - 1)) // BT) * BT
    er = jax.lax.broadcasted_iota(jnp.int32, (E, E), 0)
    ec = jax.lax.broadcasted_iota(jnp.int32, (E, E), 1)
    inc_mask = (er <= ec).astype(jnp.int32)
    cum_pc = jnp.sum(
        jnp.broadcast_to(pc.reshape(E, 1), (E, E)) * inc_mask, axis=0
    ).reshape(1, E)
    pstarts = (cum_pc - pc).astype(jnp.float32)

    rank1 = jnp.sum(excl * oh1, axis=1)
    rank2 = jnp.sum(excl * oh2, axis=1)
    start1 = jnp.sum(jnp.broadcast_to(pstarts, (T, E)) * oh1, axis=1)
    start2 = jnp.sum(jnp.broadcast_to(pstarts, (T, E)) * oh2, axis=1)
    pos1 = (start1 + rank1).astype(jnp.int32)
    pos2 = (start2 + rank2).astype(jnp.int32)
    pos_ref[...] = jnp.concatenate([pos1[:, None], pos2[:, None]], axis=1)

    nbi = jax.lax.broadcasted_iota(jnp.int32, (NBLK, E), 0) * BT
    cpe = jnp.broadcast_to(cum_pc, (NBLK, E))
    blke = jnp.sum((nbi >= cpe).astype(jnp.int32), axis=1)
    blke_ref[...] = jnp.minimum(blke, E - 1)[:, None]
    total = jnp.broadcast_to(cum_pc[:, E - 1 :], (NBLK, 1))
    nb1 = jax.lax.broadcasted_iota(jnp.int32, (NBLK, 1), 0) * BT
    blku_ref[...] = (nb1 < total).astype(jnp.int32)


def _routing(x_flat, gate_w, gate_b):
    return pl.pallas_call(
        _routing_body,
        grid=(1,),
        in_specs=[
            pl.BlockSpec((T, H), lambda i: (0, 0)),
            pl.BlockSpec((H, E), lambda i: (0, 0)),
            pl.BlockSpec((1, E), lambda i: (0, 0)),
        ],
        out_specs=[
            pl.BlockSpec((T, K), lambda i: (0, 0)),
            pl.BlockSpec((T, K), lambda i: (0, 0)),
            pl.BlockSpec((NBLK, 1), lambda i: (0, 0)),
            pl.BlockSpec((NBLK, 1), lambda i: (0, 0)),
        ],
        out_shape=[
            jax.ShapeDtypeStruct((T, K), jnp.float32),
            jax.ShapeDtypeStruct((T, K), jnp.int32),
            jax.ShapeDtypeStruct((NBLK, 1), jnp.int32),
            jax.ShapeDtypeStruct((NBLK, 1), jnp.int32),
        ],
    )(x_flat, gate_w, gate_b.reshape(1, E))


NW = 32
CS = T // NW


def _sc_dispatch(x, i0, i1):
    @pl.kernel(
        out_type=jax.ShapeDtypeStruct((NPAD, H), jnp.float32),
        mesh=_vector_mesh(),
        scratch_types=[
            pltpu.VMEM((CS,), jnp.int32),
            pltpu.VMEM((CS,), jnp.int32),
            pltpu.VMEM((CS, H), jnp.float32),
            pltpu.SemaphoreType.DMA,
            pltpu.SemaphoreType.DMA,
        ],
    )
    def sc_scatter(x_hbm, i0_hbm, i1_hbm, xs_hbm, i0_v, i1_v, x_v, sem0, sem1):
        wid = jax.lax.axis_index("s") * 2 + jax.lax.axis_index("c")
        base = wid * CS
        pltpu.sync_copy(x_hbm.at[pl.ds(base, CS)], x_v)
        pltpu.sync_copy(i0_hbm.at[pl.ds(base, CS)], i0_v)
        pltpu.sync_copy(i1_hbm.at[pl.ds(base, CS)], i1_v)
        c0 = pltpu.async_copy(x_v, xs_hbm.at[i0_v], sem0)
        c1 = pltpu.async_copy(x_v, xs_hbm.at[i1_v], sem1)
        c0.wait()
        c1.wait()

    return sc_scatter(x, i0, i1)


NF = 4
FB = F // NF


def _ffn_body(be_ref, bu_ref, xs_ref, w1_ref, b1_ref, w2_ref, b2_ref, ys_ref):
    nb = pl.program_id(0)
    nf = pl.program_id(1)

    @pl.when(bu_ref[nb] == 1)
    def _():
        xb = xs_ref[...]
        h = (
            jnp.dot(
                xb.astype(jnp.bfloat16),
                w1_ref[0].astype(jnp.bfloat16),
                preferred_element_type=jnp.float32,
            )
            + b1_ref[0]
        )
        h = 0.5 * h * (1.0 + jax.lax.erf(h * 0.7071067811865476))
        part = jnp.dot(
            h.astype(jnp.bfloat16),
            w2_ref[0].astype(jnp.bfloat16),
            preferred_element_type=jnp.float32,
        )

        @pl.when(nf == 0)
        def _():
            ys_ref[...] = part + b2_ref[0]

        @pl.when(nf != 0)
        def _():
            ys_ref[...] += part


def _ffn(xs, w1, b1, w2, b2, blk_e, blk_u):
    grid_spec = pltpu.PrefetchScalarGridSpec(
        num_scalar_prefetch=2,
        grid=(NBLK, NF),
        in_specs=[
            pl.BlockSpec((BT, H), lambda nb, nf, be, bu: (nb, 0)),
            pl.BlockSpec((1, H, FB), lambda nb, nf, be, bu: (be[nb], 0, nf)),
            pl.BlockSpec((1, 1, FB), lambda nb, nf, be, bu: (be[nb], 0, nf)),
            pl.BlockSpec((1, FB, H), lambda nb, nf, be, bu: (be[nb], nf, 0)),
            pl.BlockSpec((1, 1, H), lambda nb, nf, be, bu: (be[nb], 0, 0)),
        ],
        out_specs=pl.BlockSpec((BT, H), lambda nb, nf, be, bu: (nb, 0)),
    )
    return pl.pallas_call(
        _ffn_body,
        grid_spec=grid_spec,
        out_shape=jax.ShapeDtypeStruct((NPAD, H), jnp.float32),
    )(blk_e, blk_u, xs, w1, b1.reshape(E, 1, F), w2, b2.reshape(E, 1, H))


CG = A // NW


def _sc_combine_gather(ys, iflat):
    @pl.kernel(
        out_type=jax.ShapeDtypeStruct((A, H), jnp.float32),
        mesh=_vector_mesh(),
        scratch_types=[
            pltpu.VMEM((CG,), jnp.int32),
            pltpu.VMEM((CG, H), jnp.float32),
            pltpu.SemaphoreType.DMA,
        ],
    )
    def sc_gather(ys_hbm, i_hbm, o_hbm, i_v, r_v, sem):
        wid = jax.lax.axis_index("s") * 2 + jax.lax.axis_index("c")
        base = wid * CG
        pltpu.sync_copy(i_hbm.at[pl.ds(base, CG)], i_v)
        pltpu.async_copy(ys_hbm.at[i_v], r_v, sem).wait()
        pltpu.sync_copy(r_v, o_hbm.at[pl.ds(base, CG)])

    return sc_gather(ys, iflat)


def _combine_body(s0_ref, s1_ref, wgt_ref, out_ref):
    w = wgt_ref[...]
    out_ref[...] = w[:, 0:1] * s0_ref[...] + w[:, 1:2] * s1_ref[...]


def _combine(sel, wgt):
    return pl.pallas_call(
        _combine_body,
        grid=(T // BTC,),
        in_specs=[
            pl.BlockSpec((BTC, H), lambda i: (i, 0)),
            pl.BlockSpec((BTC, H), lambda i: (i + T // BTC, 0)),
            pl.BlockSpec((BTC, K), lambda i: (i, 0)),
        ],
        out_specs=pl.BlockSpec((BTC, H), lambda i: (i, 0)),
        out_shape=jax.ShapeDtypeStruct((T, H), jnp.float32),
    )(sel, sel, wgt)


def kernel(x, gate_w, gate_b, w1, b1, w2, b2):
    B, S, Hx = x.shape
    x_flat = x.reshape(T, Hx)

    wgt, pos, blk_e, blk_u = _routing(x_flat, gate_w, gate_b)
    i0 = pos[:, 0]
    i1 = pos[:, 1]
    iflat = jnp.concatenate([i0, i1])
    xs = _sc_dispatch(x_flat, i0, i1)
    ys = _ffn(xs, w1, b1, w2, b2, blk_e.reshape(NBLK), blk_u.reshape(NBLK))
    sel = _sc_combine_gather(ys, iflat)
    out = _combine(sel, wgt)
    return out.reshape(B, S, Hx)


# speedup vs baseline: 1.4171x; 1.4171x over previous
import jax
import jax.numpy as jnp
from jax.experimental import pallas as pl
from jax.experimental.pallas import tpu as pltpu
from jax.experimental.pallas import tpu_sc as plsc

H = 768
F = 3072
E = 8
K = 2
T = 2048
A = T * K
BT = 256
NBLK = A // BT + E
NPAD = NBLK * BT
CH = 256
BTC = 512

def _vector_mesh():
    return plsc.VectorSubcoreMesh(core_axis_name="c", subcore_axis_name="s")


def _routing_body(x_ref, gw_ref, gb_ref, wgt_ref, pos_ref, blke_ref, blku_ref):
    x = x_ref[...]
    scores = (
        jnp.dot(x, gw_ref[...], preferred_element_type=jnp.float32)
        + gb_ref[...]
    )
    lane = jax.lax.broadcasted_iota(jnp.int32, (T, E), 1)
    idx1 = jnp.argmax(scores, axis=1)
    s1 = jnp.max(scores, axis=1)
    masked = jnp.where(lane == idx1[:, None], -jnp.inf, scores)
    idx2 = jnp.argmax(masked, axis=1)
    s2 = jnp.max(masked, axis=1)
    p1 = 1.0 / (1.0 + jnp.exp(s2 - s1))
    wgt_ref[...] = jnp.concatenate(
        [p1[:, None], (1.0 - p1)[:, None]], axis=1
    )

    oh1 = (lane == idx1[:, None]).astype(jnp.float32)
    oh2 = (lane == idx2[:, None]).astype(jnp.float32)
    sel = oh1 + oh2

    r = jax.lax.broadcasted_iota(jnp.int32, (CH, CH), 0)
    c = jax.lax.broadcasted_iota(jnp.int32, (CH, CH), 1)
    tri = (c < r).astype(jnp.float32)
    excl_chunks = []
    carry = jnp.zeros((1, E), jnp.float32)
    for ci in range(T // CH):
        blk = sel[ci * CH : (ci + 1) * CH, :]
        excl_chunks.append(
            jnp.dot(tri, blk, preferred_element_type=jnp.float32) + carry
        )
        carry = carry + jnp.sum(blk, axis=0, keepdims=True)
    excl = jnp.concatenate(excl_chunks, axis=0)
    counts = carry

    ci_ = counts.astype(jnp.int32)
    pc = ((ci_ + (BT - 1)) // BT) * BT
    er = jax.lax.broadcasted_iota(jnp.int32, (E, E), 0)
    ec = jax.lax.broadcasted_iota(jnp.int32, (E, E), 1)
    inc_mask = (er <= ec).astype(jnp.int32)
    cum_pc = jnp.sum(
        jnp.broadcast_to(pc.reshape(E, 1), (E, E)) * inc_mask, axis=0
    ).reshape(1, E)
    pstarts = (cum_pc - pc).astype(jnp.float32)

    rank1 = jnp.sum(excl * oh1, axis=1)
    rank2 = jnp.sum(excl * oh2, axis=1)
    start1 = jnp.sum(jnp.broadcast_to(pstarts, (T, E)) * oh1, axis=1)
    start2 = jnp.sum(jnp.broadcast_to(pstarts, (T, E)) * oh2, axis=1)
    pos1 = (start1 + rank1).astype(jnp.int32)
    pos2 = (start2 + rank2).astype(jnp.int32)
    pos_ref[...] = jnp.concatenate([pos1[:, None], pos2[:, None]], axis=1)

    nbi = jax.lax.broadcasted_iota(jnp.int32, (NBLK, E), 0) * BT
    cpe = jnp.broadcast_to(cum_pc, (NBLK, E))
    blke = jnp.sum((nbi >= cpe).astype(jnp.int32), axis=1)
    blke_ref[...] = jnp.minimum(blke, E - 1)[:, None]
    total = jnp.broadcast_to(cum_pc[:, E - 1 :], (NBLK, 1))
    nb1 = jax.lax.broadcasted_iota(jnp.int32, (NBLK, 1), 0) * BT
    blku_ref[...] = (nb1 < total).astype(jnp.int32)


def _routing(x_flat, gate_w, gate_b):
    return pl.pallas_call(
        _routing_body,
        grid=(1,),
        in_specs=[
            pl.BlockSpec((T, H), lambda i: (0, 0)),
            pl.BlockSpec((H, E), lambda i: (0, 0)),
            pl.BlockSpec((1, E), lambda i: (0, 0)),
        ],
        out_specs=[
            pl.BlockSpec((T, K), lambda i: (0, 0)),
            pl.BlockSpec((T, K), lambda i: (0, 0)),
            pl.BlockSpec((NBLK, 1), lambda i: (0, 0)),
            pl.BlockSpec((NBLK, 1), lambda i: (0, 0)),
        ],
        out_shape=[
            jax.ShapeDtypeStruct((T, K), jnp.float32),
            jax.ShapeDtypeStruct((T, K), jnp.int32),
            jax.ShapeDtypeStruct((NBLK, 1), jnp.int32),
            jax.ShapeDtypeStruct((NBLK, 1), jnp.int32),
        ],
    )(x_flat, gate_w, gate_b.reshape(1, E))


NW = 32
CS = T // NW


def _sc_dispatch(x, i0, i1):
    @pl.kernel(
        out_type=jax.ShapeDtypeStruct((NPAD, H), jnp.float32),
        mesh=_vector_mesh(),
        scratch_types=[
            pltpu.VMEM((CS,), jnp.int32),
            pltpu.VMEM((CS,), jnp.int32),
            pltpu.VMEM((CS, H), jnp.float32),
            pltpu.SemaphoreType.DMA,
            pltpu.SemaphoreType.DMA,
        ],
    )
    def sc_scatter(x_hbm, i0_hbm, i1_hbm, xs_hbm, i0_v, i1_v, x_v, sem0, sem1):
        wid = jax.lax.axis_index("s") * 2 + jax.lax.axis_index("c")
        base = wid * CS
        pltpu.sync_copy(x_hbm.at[pl.ds(base, CS)], x_v)
        pltpu.sync_copy(i0_hbm.at[pl.ds(base, CS)], i0_v)
        pltpu.sync_copy(i1_hbm.at[pl.ds(base, CS)], i1_v)
        c0 = pltpu.async_copy(x_v, xs_hbm.at[i0_v], sem0)
        c1 = pltpu.async_copy(x_v, xs_hbm.at[i1_v], sem1)
        c0.wait()
        c1.wait()

    return sc_scatter(x, i0, i1)


def _ffn_body(
    be_ref, bu_ref, xs_ref, w1_ref, b1_ref, w2_ref, b2_ref, ys_ref, w1b, w2b
):
    nb = pl.program_id(0)
    prev = be_ref[jnp.maximum(nb - 1, 0)]
    first = jnp.logical_or(nb == 0, be_ref[nb] != prev)

    @pl.when(jnp.logical_and(bu_ref[nb] == 1, first))
    def _():
        w1b[...] = w1_ref[0].astype(jnp.bfloat16)
        w2b[...] = w2_ref[0].astype(jnp.bfloat16)

    @pl.when(bu_ref[nb] == 1)
    def _():
        xb = xs_ref[...]
        h = (
            jnp.dot(
                xb.astype(jnp.bfloat16),
                w1b[...],
                preferred_element_type=jnp.float32,
            )
            + b1_ref[0]
        )
        h = 0.5 * h * (1.0 + jax.lax.erf(h * 0.7071067811865476))
        ys_ref[...] = (
            jnp.dot(
                h.astype(jnp.bfloat16),
                w2b[...],
                preferred_element_type=jnp.float32,
            )
            + b2_ref[0]
        )


def _ffn(xs, w1, b1, w2, b2, blk_e, blk_u):
    grid_spec = pltpu.PrefetchScalarGridSpec(
        num_scalar_prefetch=2,
        grid=(NBLK,),
        in_specs=[
            pl.BlockSpec((BT, H), lambda nb, be, bu: (nb, 0)),
            pl.BlockSpec((1, H, F), lambda nb, be, bu: (be[nb], 0, 0)),
            pl.BlockSpec((1, 1, F), lambda nb, be, bu: (be[nb], 0, 0)),
            pl.BlockSpec((1, F, H), lambda nb, be, bu: (be[nb], 0, 0)),
            pl.BlockSpec((1, 1, H), lambda nb, be, bu: (be[nb], 0, 0)),
        ],
        out_specs=pl.BlockSpec((BT, H), lambda nb, be, bu: (nb, 0)),
        scratch_shapes=[
            pltpu.VMEM((H, F), jnp.bfloat16),
            pltpu.VMEM((F, H), jnp.bfloat16),
        ],
    )
    return pl.pallas_call(
        _ffn_body,
        grid_spec=grid_spec,
        out_shape=jax.ShapeDtypeStruct((NPAD, H), jnp.float32),
    )(blk_e, blk_u, xs, w1, b1.reshape(E, 1, F), w2, b2.reshape(E, 1, H))


CG = A // NW


def _sc_combine_gather(ys, iflat):
    @pl.kernel(
        out_type=jax.ShapeDtypeStruct((A, H), jnp.float32),
        mesh=_vector_mesh(),
        scratch_types=[
            pltpu.VMEM((CG,), jnp.int32),
            pltpu.VMEM((CG, H), jnp.float32),
            pltpu.SemaphoreType.DMA,
        ],
    )
    def sc_gather(ys_hbm, i_hbm, o_hbm, i_v, r_v, sem):
        wid = jax.lax.axis_index("s") * 2 + jax.lax.axis_index("c")
        base = wid * CG
        pltpu.sync_copy(i_hbm.at[pl.ds(base, CG)], i_v)
        pltpu.async_copy(ys_hbm.at[i_v], r_v, sem).wait()
        pltpu.sync_copy(r_v, o_hbm.at[pl.ds(base, CG)])

    return sc_gather(ys, iflat)


def _combine_body(s0_ref, s1_ref, wgt_ref, out_ref):
    w = wgt_ref[...]
    out_ref[...] = w[:, 0:1] * s0_ref[...] + w[:, 1:2] * s1_ref[...]


def _combine(sel, wgt):
    return pl.pallas_call(
        _combine_body,
        grid=(T // BTC,),
        in_specs=[
            pl.BlockSpec((BTC, H), lambda i: (i, 0)),
            pl.BlockSpec((BTC, H), lambda i: (i + T // BTC, 0)),
            pl.BlockSpec((BTC, K), lambda i: (i, 0)),
        ],
        out_specs=pl.BlockSpec((BTC, H), lambda i: (i, 0)),
        out_shape=jax.ShapeDtypeStruct((T, H), jnp.float32),
    )(sel, sel, wgt)


def kernel(x, gate_w, gate_b, w1, b1, w2, b2):
    B, S, Hx = x.shape
    x_flat = x.reshape(T, Hx)

    wgt, pos, blk_e, blk_u = _routing(x_flat, gate_w, gate_b)
    i0 = pos[:, 0]
    i1 = pos[:, 1]
    iflat = jnp.concatenate([i0, i1])
    xs = _sc_dispatch(x_flat, i0, i1)
    ys = _ffn(xs, w1, b1, w2, b2, blk_e.reshape(NBLK), blk_u.reshape(NBLK))
    sel = _sc_combine_gather(ys, iflat)
    out = _combine(sel, wgt)
    return out.reshape(B, S, Hx)


# speedup vs baseline: 1.7008x; 1.2001x over previous
import jax
import jax.numpy as jnp
from jax.experimental import pallas as pl
from jax.experimental.pallas import tpu as pltpu
from jax.experimental.pallas import tpu_sc as plsc

H = 768
F = 3072
E = 8
K = 2
T = 2048
A = T * K
BT = 256
NBLK = A // BT + E
NPAD = NBLK * BT
CH = 256
BTC = 512

def _vector_mesh():
    return plsc.VectorSubcoreMesh(core_axis_name="c", subcore_axis_name="s")


def _routing_body(x_ref, gw_ref, gb_ref, wgt_ref, pos_ref, blke_ref, blku_ref):
    x = x_ref[...]
    scores = (
        jnp.dot(x, gw_ref[...], preferred_element_type=jnp.float32)
        + gb_ref[...]
    )
    lane = jax.lax.broadcasted_iota(jnp.int32, (T, E), 1)
    idx1 = jnp.argmax(scores, axis=1)
    s1 = jnp.max(scores, axis=1)
    masked = jnp.where(lane == idx1[:, None], -jnp.inf, scores)
    idx2 = jnp.argmax(masked, axis=1)
    s2 = jnp.max(masked, axis=1)
    p1 = 1.0 / (1.0 + jnp.exp(s2 - s1))
    wgt_ref[...] = jnp.concatenate(
        [p1[:, None], (1.0 - p1)[:, None]], axis=1
    )

    oh1 = (lane == idx1[:, None]).astype(jnp.float32)
    oh2 = (lane == idx2[:, None]).astype(jnp.float32)
    sel = oh1 + oh2

    r = jax.lax.broadcasted_iota(jnp.int32, (CH, CH), 0)
    c = jax.lax.broadcasted_iota(jnp.int32, (CH, CH), 1)
    tri = (c < r).astype(jnp.float32)
    excl_chunks = []
    carry = jnp.zeros((1, E), jnp.float32)
    for ci in range(T // CH):
        blk = sel[ci * CH : (ci + 1) * CH, :]
        excl_chunks.append(
            jnp.dot(tri, blk, preferred_element_type=jnp.float32) + carry
        )
        carry = carry + jnp.sum(blk, axis=0, keepdims=True)
    excl = jnp.concatenate(excl_chunks, axis=0)
    counts = carry

    ci_ = counts.astype(jnp.int32)
    pc = ((ci_ + (BT - 1)) // BT) * BT
    er = jax.lax.broadcasted_iota(jnp.int32, (E, E), 0)
    ec = jax.lax.broadcasted_iota(jnp.int32, (E, E), 1)
    inc_mask = (er <= ec).astype(jnp.int32)
    cum_pc = jnp.sum(
        jnp.broadcast_to(pc.reshape(E, 1), (E, E)) * inc_mask, axis=0
    ).reshape(1, E)
    pstarts = (cum_pc - pc).astype(jnp.float32)

    rank1 = jnp.sum(excl * oh1, axis=1)
    rank2 = jnp.sum(excl * oh2, axis=1)
    start1 = jnp.sum(jnp.broadcast_to(pstarts, (T, E)) * oh1, axis=1)
    start2 = jnp.sum(jnp.broadcast_to(pstarts, (T, E)) * oh2, axis=1)
    pos1 = (start1 + rank1).astype(jnp.int32)
    pos2 = (start2 + rank2).astype(jnp.int32)
    pos_ref[...] = jnp.concatenate([pos1[:, None], pos2[:, None]], axis=1)

    nbi = jax.lax.broadcasted_iota(jnp.int32, (NBLK, E), 0) * BT
    cpe = jnp.broadcast_to(cum_pc, (NBLK, E))
    blke = jnp.sum((nbi >= cpe).astype(jnp.int32), axis=1)
    blke_ref[...] = jnp.minimum(blke, E - 1)[:, None]
    total = jnp.broadcast_to(cum_pc[:, E - 1 :], (NBLK, 1))
    nb1 = jax.lax.broadcasted_iota(jnp.int32, (NBLK, 1), 0) * BT
    blku_ref[...] = (nb1 < total).astype(jnp.int32)


def _routing(x_flat, gate_w, gate_b):
    return pl.pallas_call(
        _routing_body,
        grid=(1,),
        in_specs=[
            pl.BlockSpec((T, H), lambda i: (0, 0)),
            pl.BlockSpec((H, E), lambda i: (0, 0)),
            pl.BlockSpec((1, E), lambda i: (0, 0)),
        ],
        out_specs=[
            pl.BlockSpec((T, K), lambda i: (0, 0)),
            pl.BlockSpec((T, K), lambda i: (0, 0)),
            pl.BlockSpec((NBLK, 1), lambda i: (0, 0)),
            pl.BlockSpec((NBLK, 1), lambda i: (0, 0)),
        ],
        out_shape=[
            jax.ShapeDtypeStruct((T, K), jnp.float32),
            jax.ShapeDtypeStruct((T, K), jnp.int32),
            jax.ShapeDtypeStruct((NBLK, 1), jnp.int32),
            jax.ShapeDtypeStruct((NBLK, 1), jnp.int32),
        ],
    )(x_flat, gate_w, gate_b.reshape(1, E))


NW = 32
CS = T // NW


def _sc_dispatch(x, i0, i1):
    @pl.kernel(
        out_type=jax.ShapeDtypeStruct((NPAD, H), jnp.float32),
        mesh=_vector_mesh(),
        scratch_types=[
            pltpu.VMEM((CS,), jnp.int32),
            pltpu.VMEM((CS,), jnp.int32),
            pltpu.VMEM((CS, H), jnp.float32),
            pltpu.SemaphoreType.DMA,
            pltpu.SemaphoreType.DMA,
        ],
    )
    def sc_scatter(x_hbm, i0_hbm, i1_hbm, xs_hbm, i0_v, i1_v, x_v, sem0, sem1):
        wid = jax.lax.axis_index("s") * 2 + jax.lax.axis_index("c")
        base = wid * CS
        pltpu.sync_copy(x_hbm.at[pl.ds(base, CS)], x_v)
        pltpu.sync_copy(i0_hbm.at[pl.ds(base, CS)], i0_v)
        pltpu.sync_copy(i1_hbm.at[pl.ds(base, CS)], i1_v)
        c0 = pltpu.async_copy(x_v, xs_hbm.at[i0_v], sem0)
        c1 = pltpu.async_copy(x_v, xs_hbm.at[i1_v], sem1)
        c0.wait()
        c1.wait()

    return sc_scatter(x, i0, i1)


def _ffn_body(
    be_ref, bu_ref, fi_ref, eo_ref, nx_ref, hx_ref,
    xs_ref, w1_hbm, b1_ref, w2_hbm, b2_ref, ys_ref, w1s, w2s, sems,
):
    nb = pl.program_id(0)
    e = be_ref[nb]
    slot = jax.lax.rem(eo_ref[nb], 2)

    def stage(s):
        @pl.when(fi_ref[nb] == 1)
        def _():
            @pl.when(nb == 0)
            def _():
                pltpu.make_async_copy(w1_hbm.at[e], w1s.at[s], sems.at[s, 0]).start()
                pltpu.make_async_copy(w2_hbm.at[e], w2s.at[s], sems.at[s, 1]).start()

            pltpu.make_async_copy(w1_hbm.at[e], w1s.at[s], sems.at[s, 0]).wait()
            pltpu.make_async_copy(w2_hbm.at[e], w2s.at[s], sems.at[s, 1]).wait()

            @pl.when(hx_ref[nb] == 1)
            def _():
                nx = nx_ref[nb]
                pltpu.make_async_copy(
                    w1_hbm.at[nx], w1s.at[1 - s], sems.at[1 - s, 0]
                ).start()
                pltpu.make_async_copy(
                    w2_hbm.at[nx], w2s.at[1 - s], sems.at[1 - s, 1]
                ).start()

        @pl.when(bu_ref[nb] == 1)
        def _():
            xb = xs_ref[...]
            h = (
                jnp.dot(
                    xb.astype(jnp.bfloat16),
                    w1s[s].astype(jnp.bfloat16),
                    preferred_element_type=jnp.float32,
                )
                + b1_ref[0]
            )
            h = 0.5 * h * (1.0 + jax.lax.erf(h * 0.7071067811865476))
            ys_ref[...] = (
                jnp.dot(
                    h.astype(jnp.bfloat16),
                    w2s[s].astype(jnp.bfloat16),
                    preferred_element_type=jnp.float32,
                )
                + b2_ref[0]
            )

    @pl.when(slot == 0)
    def _():
        stage(0)

    @pl.when(slot == 1)
    def _():
        stage(1)


def _ffn(xs, w1, b1, w2, b2, blk_e, blk_u, blk_fi, blk_eo, blk_nx, blk_hx):
    grid_spec = pltpu.PrefetchScalarGridSpec(
        num_scalar_prefetch=6,
        grid=(NBLK,),
        in_specs=[
            pl.BlockSpec((BT, H), lambda nb, *_: (nb, 0)),
            pl.BlockSpec(memory_space=pltpu.MemorySpace.HBM),
            pl.BlockSpec((1, 1, F), lambda nb, be, *_: (be[nb], 0, 0)),
            pl.BlockSpec(memory_space=pltpu.MemorySpace.HBM),
            pl.BlockSpec((1, 1, H), lambda nb, be, *_: (be[nb], 0, 0)),
        ],
        out_specs=pl.BlockSpec((BT, H), lambda nb, *_: (nb, 0)),
        scratch_shapes=[
            pltpu.VMEM((2, H, F), jnp.float32),
            pltpu.VMEM((2, F, H), jnp.float32),
            pltpu.SemaphoreType.DMA((2, 2)),
        ],
    )
    return pl.pallas_call(
        _ffn_body,
        grid_spec=grid_spec,
        out_shape=jax.ShapeDtypeStruct((NPAD, H), jnp.float32),
    )(
        blk_e, blk_u, blk_fi, blk_eo, blk_nx, blk_hx,
        xs, w1, b1.reshape(E, 1, F), w2, b2.reshape(E, 1, H),
    )


CG = A // NW


def _sc_combine_gather(ys, iflat):
    @pl.kernel(
        out_type=jax.ShapeDtypeStruct((A, H), jnp.float32),
        mesh=_vector_mesh(),
        scratch_types=[
            pltpu.VMEM((CG,), jnp.int32),
            pltpu.VMEM((CG, H), jnp.float32),
            pltpu.SemaphoreType.DMA,
        ],
    )
    def sc_gather(ys_hbm, i_hbm, o_hbm, i_v, r_v, sem):
        wid = jax.lax.axis_index("s") * 2 + jax.lax.axis_index("c")
        base = wid * CG
        pltpu.sync_copy(i_hbm.at[pl.ds(base, CG)], i_v)
        pltpu.async_copy(ys_hbm.at[i_v], r_v, sem).wait()
        pltpu.sync_copy(r_v, o_hbm.at[pl.ds(base, CG)])

    return sc_gather(ys, iflat)


def _combine_body(s0_ref, s1_ref, wgt_ref, out_ref):
    w = wgt_ref[...]
    out_ref[...] = w[:, 0:1] * s0_ref[...] + w[:, 1:2] * s1_ref[...]


def _combine(sel, wgt):
    return pl.pallas_call(
        _combine_body,
        grid=(T // BTC,),
        in_specs=[
            pl.BlockSpec((BTC, H), lambda i: (i, 0)),
            pl.BlockSpec((BTC, H), lambda i: (i + T // BTC, 0)),
            pl.BlockSpec((BTC, K), lambda i: (i, 0)),
        ],
        out_specs=pl.BlockSpec((BTC, H), lambda i: (i, 0)),
        out_shape=jax.ShapeDtypeStruct((T, H), jnp.float32),
    )(sel, sel, wgt)


def kernel(x, gate_w, gate_b, w1, b1, w2, b2):
    B, S, Hx = x.shape
    x_flat = x.reshape(T, Hx)

    wgt, pos, blk_e, blk_u = _routing(x_flat, gate_w, gate_b)
    i0 = pos[:, 0]
    i1 = pos[:, 1]
    iflat = jnp.concatenate([i0, i1])

    be = blk_e.reshape(NBLK)
    bu = blk_u.reshape(NBLK)
    fi = (
        jnp.concatenate(
            [jnp.ones((1,), jnp.int32), (be[1:] != be[:-1]).astype(jnp.int32)]
        )
        * bu
    )
    eo = jnp.cumsum(fi) - 1
    io = jnp.arange(NBLK, dtype=jnp.int32)
    nxi = jnp.min(
        jnp.where((io[None, :] > io[:, None]) & (fi[None, :] == 1),
                  io[None, :], NBLK),
        axis=1,
    )
    hx = (nxi < NBLK).astype(jnp.int32)
    nx = be[jnp.clip(nxi, 0, NBLK - 1)]

    xs = _sc_dispatch(x_flat, i0, i1)
    ys = _ffn(xs, w1, b1, w2, b2, be, bu, fi, eo, nx, hx)
    sel = _sc_combine_gather(ys, iflat)
    out = _combine(sel, wgt)
    return out.reshape(B, S, Hx)
